# baseline (speedup 1.0000x reference)
import functools

import jax
import jax.numpy as jnp
import numpy as np
from jax import lax
from jax.experimental import pallas as pl
from jax.experimental.pallas import tpu as pltpu
from jax.experimental.pallas import tpu_sc as plsc

H = 16
S = 2048
NB = 32
SHIFTS = 128
PW = 3968
LW = 4096

_BLOCKS_PER_HEAD = S // SHIFTS
_BLOCKS_PER_WORKER = H * _BLOCKS_PER_HEAD // 32
_LOG16 = np.float32(np.log(np.float64(16.0)))


def _pack_tc_body(tt_ref, p_ref):
    u = lax.broadcasted_iota(jnp.int32, (H, LW), 1)
    d = u - (S - 1)
    neg = d < 0
    ad = jnp.abs(d)
    is_small = ad < 8
    x = ad.astype(jnp.float32) / jnp.float32(8.0)
    vlarge = 8 + (jnp.log(x) / _LOG16 * jnp.float32(8.0)).astype(jnp.int32)
    vlarge = jnp.minimum(vlarge, 15)
    bucket = jnp.where(neg, 16, 0) + jnp.where(is_small, ad, vlarge)
    line = jnp.zeros((H, LW), jnp.float32)
    for b in range(NB):
        line = jnp.where(bucket == b, tt_ref[:, b : b + 1], line)
    riota = lax.broadcasted_iota(jnp.int32, (SHIFTS, LW), 0)
    shift_amt = (SHIFTS - 1) - riota
    for h in range(H):
        work = jnp.broadcast_to(line[h : h + 1, :], (SHIFTS, LW))
        for k in range(6, -1, -1):
            step = 1 << k
            rolled = pltpu.roll(work, LW - step, axis=1)
            take = ((shift_amt >> k) & 1) == 1
            work = jnp.where(take, rolled, work)
        p_ref[h, :, :] = work[:, :PW]


_pack_tc = pl.pallas_call(
    _pack_tc_body,
    out_shape=jax.ShapeDtypeStruct((H, SHIFTS, PW), jnp.float32),
)


@functools.partial(
    pl.kernel,
    out_type=jax.ShapeDtypeStruct((1, H, S, S), jnp.float32),
    mesh=plsc.VectorSubcoreMesh(core_axis_name="c", subcore_axis_name="s"),
    compiler_params=pltpu.CompilerParams(use_tc_tiling_on_sc=True),
)
def _expand_sc(p_hbm, out_hbm):
    info = plsc.get_sparse_core_info()
    nc = info.num_cores
    wid = lax.axis_index("s") * nc + lax.axis_index("c")

    def body(j, carry):
        blk = wid * _BLOCKS_PER_WORKER + j
        h = blk // _BLOCKS_PER_HEAD
        q = blk % _BLOCKS_PER_HEAD
        base = SHIFTS * (_BLOCKS_PER_HEAD - 1 - q)
        pltpu.sync_copy(
            p_hbm.at[h, :, pl.ds(base, S)],
            out_hbm.at[0, h, pl.ds(SHIFTS * q, SHIFTS)],
        )
        return carry

    lax.fori_loop(0, _BLOCKS_PER_WORKER, body, 0)


def kernel(qlen, klen, rel_bias_table):
    tt = jnp.transpose(rel_bias_table)
    p_all = _pack_tc(tt)
    return _expand_sc(p_all)


# speedup vs baseline: 21.3706x; 21.3706x over previous
import functools

import jax
import jax.numpy as jnp
import numpy as np
from jax import lax
from jax.experimental import pallas as pl
from jax.experimental.pallas import tpu as pltpu
from jax.experimental.pallas import tpu_sc as plsc

H = 16
S = 2048
NB = 32
SHIFTS = 16
PW = 4224
LW = 4352

_BLOCKS_PER_HEAD = S // SHIFTS
_LOG16 = np.float32(np.log(np.float64(16.0)))


def _line_tc_body(tt_ref, p_ref):
    u = lax.broadcasted_iota(jnp.int32, (H, LW), 1)
    d = u - (S - 1)
    neg = d < 0
    ad = jnp.abs(d)
    is_small = ad < 8
    x = ad.astype(jnp.float32) / jnp.float32(8.0)
    vlarge = 8 + (jnp.log(x) / _LOG16 * jnp.float32(8.0)).astype(jnp.int32)
    vlarge = jnp.minimum(vlarge, 15)
    bucket = jnp.where(neg, 16, 0) + jnp.where(is_small, ad, vlarge)
    line = jnp.zeros((H, LW), jnp.float32)
    for b in range(NB):
        line = jnp.where(bucket == b, tt_ref[:, b : b + 1], line)
    for r in range(SHIFTS):
        sh = SHIFTS - 1 - r
        p_ref[:, r, :] = lax.slice(line, (0, sh), (H, sh + PW))


_line_tc = pl.pallas_call(
    _line_tc_body,
    out_shape=jax.ShapeDtypeStruct((H, SHIFTS, PW), jnp.float32),
)


@functools.partial(
    pl.kernel,
    out_type=jax.ShapeDtypeStruct((1, H, S, S), jnp.float32),
    mesh=plsc.VectorSubcoreMesh(core_axis_name="c", subcore_axis_name="s"),
    scratch_types=[
        pltpu.VMEM((SHIFTS, PW), jnp.float32),
        pltpu.SemaphoreType.DMA,
    ],
    compiler_params=pltpu.CompilerParams(use_tc_tiling_on_sc=False),
)
def _expand_sc(p_hbm, out_hbm, p_v, sem):
    info = plsc.get_sparse_core_info()
    nc = info.num_cores
    wid = lax.axis_index("s") * nc + lax.axis_index("c")
    h = wid // 2
    half = wid % 2
    pltpu.sync_copy(p_hbm.at[h], p_v)
    q0 = half * (_BLOCKS_PER_HEAD // 2)

    descs = []
    for q in range(_BLOCKS_PER_HEAD // 2):
        qq = q0 + q
        base = SHIFTS * (_BLOCKS_PER_HEAD - 1 - qq)
        descs.append(
            pltpu.async_copy(
                p_v.at[:, pl.ds(base, S)],
                out_hbm.at[0, h, pl.ds(SHIFTS * qq, SHIFTS)],
                sem,
            )
        )
    for d in descs:
        d.wait()


def kernel(qlen, klen, rel_bias_table):
    tt = jnp.transpose(rel_bias_table)
    p_all = _line_tc(tt)
    return _expand_sc(p_all)


# speedup vs baseline: 33.3489x; 1.5605x over previous
import functools

import jax
import jax.numpy as jnp
import numpy as np
from jax import lax
from jax.experimental import pallas as pl
from jax.experimental.pallas import tpu as pltpu
from jax.experimental.pallas import tpu_sc as plsc

H = 16
S = 2048
NB = 32
SHIFTS = 128
PW = 3968
LW = 4096
WAVE_H = 2

_BLOCKS_PER_HEAD = S // SHIFTS
_HEADS_PER_SC = H // 2
_WAVES = _HEADS_PER_SC // WAVE_H
_COPIES_PER_WAVE = WAVE_H * _BLOCKS_PER_HEAD
_STAGE_ROWS = WAVE_H * SHIFTS // 16
_LOG16 = np.float32(np.log(np.float64(16.0)))


def _pack_tc_body(tt_ref, p_ref):
    u = lax.broadcasted_iota(jnp.int32, (H, LW), 1)
    d = u - (S - 1)
    neg = d < 0
    ad = jnp.abs(d)
    is_small = ad < 8
    x = ad.astype(jnp.float32) / jnp.float32(8.0)
    vlarge = 8 + (jnp.log(x) / _LOG16 * jnp.float32(8.0)).astype(jnp.int32)
    vlarge = jnp.minimum(vlarge, 15)
    bucket = jnp.where(neg, 16, 0) + jnp.where(is_small, ad, vlarge)
    line = jnp.zeros((H, LW), jnp.float32)
    for b in range(NB):
        line = jnp.where(bucket == b, tt_ref[:, b : b + 1], line)
    riota = lax.broadcasted_iota(jnp.int32, (SHIFTS, LW), 0)
    shift_amt = (SHIFTS - 1) - riota
    for h in range(H):
        work = jnp.broadcast_to(line[h : h + 1, :], (SHIFTS, LW))
        for k in range(6, -1, -1):
            step = 1 << k
            rolled = pltpu.roll(work, LW - step, axis=1)
            take = ((shift_amt >> k) & 1) == 1
            work = jnp.where(take, rolled, work)
        p_ref[h, :, :] = work[:, :PW]


_pack_tc = pl.pallas_call(
    _pack_tc_body,
    out_shape=jax.ShapeDtypeStruct((H, SHIFTS, PW), jnp.float32),
)


@functools.partial(
    pl.kernel,
    out_type=jax.ShapeDtypeStruct((1, H, S, S), jnp.float32),
    mesh=plsc.VectorSubcoreMesh(core_axis_name="c", subcore_axis_name="s"),
    scratch_types=[
        pltpu.VMEM_SHARED((WAVE_H, SHIFTS, PW), jnp.float32),
        pltpu.SemaphoreType.DMA,
    ],
    compiler_params=pltpu.CompilerParams(use_tc_tiling_on_sc=True),
)
def _expand_sc(p_hbm, out_hbm, shared, sem):
    sc = lax.axis_index("c")
    sid = lax.axis_index("s")
    h0 = sc * _HEADS_PER_SC

    for wave in range(_WAVES):
        hw = h0 + wave * WAVE_H
        hl = sid // (16 // WAVE_H)
        r0 = (sid % (16 // WAVE_H)) * _STAGE_ROWS
        pltpu.sync_copy(
            p_hbm.at[hw + hl, pl.ds(r0, _STAGE_ROWS), :],
            shared.at[hl, pl.ds(r0, _STAGE_ROWS), :],
        )
        plsc.subcore_barrier()
        descs = []
        for i in range(_COPIES_PER_WAVE // 16):
            k = sid * (_COPIES_PER_WAVE // 16) + i
            khl = k // _BLOCKS_PER_HEAD
            a = k % _BLOCKS_PER_HEAD
            base = SHIFTS * (_BLOCKS_PER_HEAD - 1 - a)
            descs.append(
                pltpu.async_copy(
                    shared.at[khl, :, pl.ds(base, S)],
                    out_hbm.at[0, hw + khl, pl.ds(SHIFTS * a, SHIFTS)],
                    sem,
                )
            )
        for dsc in descs:
            dsc.wait()
        plsc.subcore_barrier()


def kernel(qlen, klen, rel_bias_table):
    tt = jnp.transpose(rel_bias_table)
    p_all = _pack_tc(tt)
    return _expand_sc(p_all)


# speedup vs baseline: 38.3639x; 1.1504x over previous
import functools

import jax
import jax.numpy as jnp
import numpy as np
from jax import lax
from jax.experimental import pallas as pl
from jax.experimental.pallas import tpu as pltpu
from jax.experimental.pallas import tpu_sc as plsc

H = 16
S = 2048
NB = 32
SHIFTS = 128
PW = 3968
LW = 4096
WAVE_H = 2

_BLOCKS_PER_HEAD = S // SHIFTS
_HEADS_PER_SC = H // 2
_WAVES = _HEADS_PER_SC // WAVE_H
_COPIES_PER_WAVE = WAVE_H * _BLOCKS_PER_HEAD
_STAGE_ROWS = WAVE_H * SHIFTS // 16
_LOG16 = np.float32(np.log(np.float64(16.0)))


def _pack_tc_body(tt_ref, p_ref):
    u = lax.broadcasted_iota(jnp.int32, (H, LW), 1)
    d = u - (S - 1)
    neg = d < 0
    ad = jnp.abs(d)
    is_small = ad < 8
    x = ad.astype(jnp.float32) / jnp.float32(8.0)
    vlarge = 8 + (jnp.log(x) / _LOG16 * jnp.float32(8.0)).astype(jnp.int32)
    vlarge = jnp.minimum(vlarge, 15)
    bucket = jnp.where(neg, 16, 0) + jnp.where(is_small, ad, vlarge)
    line = jnp.zeros((H, LW), jnp.float32)
    for b in range(NB):
        line = jnp.where(bucket == b, tt_ref[:, b : b + 1], line)
    w = pltpu.roll(line, LW - (SHIFTS - 1), axis=1)
    for r in range(SHIFTS):
        p_ref[:, r, :] = w[:, :PW]
        if r < SHIFTS - 1:
            w = pltpu.roll(w, 1, axis=1)


_pack_tc = pl.pallas_call(
    _pack_tc_body,
    out_shape=jax.ShapeDtypeStruct((H, SHIFTS, PW), jnp.float32),
)


@functools.partial(
    pl.kernel,
    out_type=jax.ShapeDtypeStruct((1, H, S, S), jnp.float32),
    mesh=plsc.VectorSubcoreMesh(core_axis_name="c", subcore_axis_name="s"),
    scratch_types=[
        pltpu.VMEM_SHARED((WAVE_H, SHIFTS, PW), jnp.float32),
        pltpu.SemaphoreType.DMA,
    ],
    compiler_params=pltpu.CompilerParams(use_tc_tiling_on_sc=True),
)
def _expand_sc(p_hbm, out_hbm, shared, sem):
    sc = lax.axis_index("c")
    sid = lax.axis_index("s")
    h0 = sc * _HEADS_PER_SC

    for wave in range(_WAVES):
        hw = h0 + wave * WAVE_H
        hl = sid // (16 // WAVE_H)
        r0 = (sid % (16 // WAVE_H)) * _STAGE_ROWS
        pltpu.sync_copy(
            p_hbm.at[hw + hl, pl.ds(r0, _STAGE_ROWS), :],
            shared.at[hl, pl.ds(r0, _STAGE_ROWS), :],
        )
        plsc.subcore_barrier()
        descs = []
        for i in range(_COPIES_PER_WAVE // 16):
            k = sid * (_COPIES_PER_WAVE // 16) + i
            khl = k // _BLOCKS_PER_HEAD
            a = k % _BLOCKS_PER_HEAD
            base = SHIFTS * (_BLOCKS_PER_HEAD - 1 - a)
            descs.append(
                pltpu.async_copy(
                    shared.at[khl, :, pl.ds(base, S)],
                    out_hbm.at[0, hw + khl, pl.ds(SHIFTS * a, SHIFTS)],
                    sem,
                )
            )
        for dsc in descs:
            dsc.wait()
        plsc.subcore_barrier()


def kernel(qlen, klen, rel_bias_table):
    tt = jnp.transpose(rel_bias_table)
    p_all = _pack_tc(tt)
    return _expand_sc(p_all)


# speedup vs baseline: 39.5362x; 1.0306x over previous
import functools

import jax
import jax.numpy as jnp
import numpy as np
from jax import lax
from jax.experimental import pallas as pl
from jax.experimental.pallas import tpu as pltpu
from jax.experimental.pallas import tpu_sc as plsc

H = 16
S = 2048
NB = 32
SHIFTS = 128
PW = 3968
LW = 4096
WAVE_H = 4

_BLOCKS_PER_HEAD = S // SHIFTS
_HEADS_PER_SC = H // 2
_WAVES = _HEADS_PER_SC // WAVE_H
_COPIES_PER_WAVE = WAVE_H * _BLOCKS_PER_HEAD
_STAGE_ROWS = WAVE_H * SHIFTS // 16
_LOG16 = np.float32(np.log(np.float64(16.0)))


def _pack_tc_body(tt_ref, p_ref):
    u = lax.broadcasted_iota(jnp.int32, (H, LW), 1)
    d = u - (S - 1)
    neg = d < 0
    ad = jnp.abs(d)
    is_small = ad < 8
    x = ad.astype(jnp.float32) / jnp.float32(8.0)
    vlarge = 8 + (jnp.log(x) / _LOG16 * jnp.float32(8.0)).astype(jnp.int32)
    vlarge = jnp.minimum(vlarge, 15)
    bucket = jnp.where(neg, 16, 0) + jnp.where(is_small, ad, vlarge)
    line = jnp.zeros((H, LW), jnp.float32)
    for b in range(NB):
        line = jnp.where(bucket == b, tt_ref[:, b : b + 1], line)
    w = pltpu.roll(line, LW - (SHIFTS - 1), axis=1)
    for r in range(SHIFTS):
        p_ref[:, r, :] = w[:, :PW]
        if r < SHIFTS - 1:
            w = pltpu.roll(w, 1, axis=1)


_pack_tc = pl.pallas_call(
    _pack_tc_body,
    out_shape=jax.ShapeDtypeStruct((H, SHIFTS, PW), jnp.float32),
)


@functools.partial(
    pl.kernel,
    out_type=jax.ShapeDtypeStruct((1, H, S, S), jnp.float32),
    mesh=plsc.VectorSubcoreMesh(core_axis_name="c", subcore_axis_name="s"),
    scratch_types=[
        pltpu.VMEM_SHARED((WAVE_H, SHIFTS, PW), jnp.float32),
        pltpu.SemaphoreType.DMA,
    ],
    compiler_params=pltpu.CompilerParams(use_tc_tiling_on_sc=True),
)
def _expand_sc(p_hbm, out_hbm, shared, sem):
    sc = lax.axis_index("c")
    sid = lax.axis_index("s")
    h0 = sc * _HEADS_PER_SC

    for wave in range(_WAVES):
        hw = h0 + wave * WAVE_H
        hl = sid // (16 // WAVE_H)
        r0 = (sid % (16 // WAVE_H)) * _STAGE_ROWS
        pltpu.sync_copy(
            p_hbm.at[hw + hl, pl.ds(r0, _STAGE_ROWS), :],
            shared.at[hl, pl.ds(r0, _STAGE_ROWS), :],
        )
        plsc.subcore_barrier()
        descs = []
        for i in range(_COPIES_PER_WAVE // 16):
            k = sid * (_COPIES_PER_WAVE // 16) + i
            khl = k // _BLOCKS_PER_HEAD
            a = k % _BLOCKS_PER_HEAD
            base = SHIFTS * (_BLOCKS_PER_HEAD - 1 - a)
            descs.append(
                pltpu.async_copy(
                    shared.at[khl, :, pl.ds(base, S)],
                    out_hbm.at[0, hw + khl, pl.ds(SHIFTS * a, SHIFTS)],
                    sem,
                )
            )
        for dsc in descs:
            dsc.wait()
        plsc.subcore_barrier()


def kernel(qlen, klen, rel_bias_table):
    tt = jnp.transpose(rel_bias_table)
    p_all = _pack_tc(tt)
    return _expand_sc(p_all)


# speedup vs baseline: 41.2398x; 1.0431x over previous
import functools

import jax
import jax.numpy as jnp
import numpy as np
from jax import lax
from jax.experimental import pallas as pl
from jax.experimental.pallas import tpu as pltpu
from jax.experimental.pallas import tpu_sc as plsc

H = 16
S = 2048
NB = 32
SHIFTS = 128
PW = 3968
LW = 4096
_BLOCKS_PER_HEAD = S // SHIFTS
_HEADS_PER_SC = H // 2
_STAGE_ROWS = SHIFTS // 16
_LOG16 = np.float32(np.log(np.float64(16.0)))


def _pack_tc_body(tt_ref, p_ref):
    u = lax.broadcasted_iota(jnp.int32, (H, LW), 1)
    d = u - (S - 1)
    neg = d < 0
    ad = jnp.abs(d)
    is_small = ad < 8
    x = ad.astype(jnp.float32) / jnp.float32(8.0)
    vlarge = 8 + (jnp.log(x) / _LOG16 * jnp.float32(8.0)).astype(jnp.int32)
    vlarge = jnp.minimum(vlarge, 15)
    bucket = jnp.where(neg, 16, 0) + jnp.where(is_small, ad, vlarge)
    line = jnp.zeros((H, LW), jnp.float32)
    for b in range(NB):
        line = jnp.where(bucket == b, tt_ref[:, b : b + 1], line)
    w = pltpu.roll(line, LW - (SHIFTS - 1), axis=1)
    for r in range(SHIFTS):
        p_ref[:, r, :] = w[:, :PW]
        if r < SHIFTS - 1:
            w = pltpu.roll(w, 1, axis=1)


_pack_tc = pl.pallas_call(
    _pack_tc_body,
    out_shape=jax.ShapeDtypeStruct((H, SHIFTS, PW), jnp.float32),
)


@functools.partial(
    pl.kernel,
    out_type=jax.ShapeDtypeStruct((1, H, S, S), jnp.float32),
    mesh=plsc.VectorSubcoreMesh(core_axis_name="c", subcore_axis_name="s"),
    scratch_types=[
        pltpu.VMEM_SHARED((2, SHIFTS, PW), jnp.float32),
        pltpu.SemaphoreType.DMA,
    ],
    compiler_params=pltpu.CompilerParams(use_tc_tiling_on_sc=True),
)
def _expand_sc(p_hbm, out_hbm, shared, sem):
    sc = lax.axis_index("c")
    sid = lax.axis_index("s")
    h0 = sc * _HEADS_PER_SC

    pltpu.sync_copy(
        p_hbm.at[h0, pl.ds(sid * _STAGE_ROWS, _STAGE_ROWS), :],
        shared.at[0, pl.ds(sid * _STAGE_ROWS, _STAGE_ROWS), :],
    )
    plsc.subcore_barrier()
    for w in range(_HEADS_PER_SC):
        cur = w % 2
        nxt = (w + 1) % 2
        stage = None
        if w + 1 < _HEADS_PER_SC:
            stage = pltpu.async_copy(
                p_hbm.at[h0 + w + 1, pl.ds(sid * _STAGE_ROWS, _STAGE_ROWS), :],
                shared.at[nxt, pl.ds(sid * _STAGE_ROWS, _STAGE_ROWS), :],
                sem,
            )
        base = SHIFTS * (_BLOCKS_PER_HEAD - 1 - sid)
        pltpu.sync_copy(
            shared.at[cur, :, pl.ds(base, S)],
            out_hbm.at[0, h0 + w, pl.ds(SHIFTS * sid, SHIFTS)],
        )
        if stage is not None:
            stage.wait()
        plsc.subcore_barrier()


def kernel(qlen, klen, rel_bias_table):
    tt = jnp.transpose(rel_bias_table)
    p_all = _pack_tc(tt)
    return _expand_sc(p_all)
